# baseline (speedup 1.0000x reference)
import functools

import jax
import jax.numpy as jnp
from jax import lax
from jax.experimental import pallas as pl
from jax.experimental.pallas import tpu as pltpu

N, E, D_FEAT, D_EDGE, EMB, H = 10000, 160000, 256, 16, 16, 1024
NPAD = 10240


def _leaky(v):
    return jnp.where(v >= 0, v, 0.01 * v)


def _h_body(x_ref, enc_ref, wx_ref, we_ref, b_ref, o_ref):
    acc = lax.dot_general(
        x_ref[...], wx_ref[...], (((1,), (0,)), ((), ())),
        preferred_element_type=jnp.float32, precision=lax.Precision.HIGHEST)
    acc += lax.dot_general(
        enc_ref[...], we_ref[...], (((1,), (0,)), ((), ())),
        preferred_element_type=jnp.float32, precision=lax.Precision.HIGHEST)
    acc += b_ref[...]
    o_ref[...] = _leaky(acc)


def _compute_h(xp, encp, W_in, b_in):
    NB = 1024
    grid = (NPAD // NB,)
    return pl.pallas_call(
        _h_body,
        grid=grid,
        in_specs=[
            pl.BlockSpec((NB, D_FEAT), lambda i: (i, 0)),
            pl.BlockSpec((NB, EMB), lambda i: (i, 0)),
            pl.BlockSpec((D_FEAT, H), lambda i: (0, 0)),
            pl.BlockSpec((EMB, H), lambda i: (0, 0)),
            pl.BlockSpec((1, H), lambda i: (0, 0)),
        ],
        out_specs=pl.BlockSpec((NB, H), lambda i: (i, 0)),
        out_shape=jax.ShapeDtypeStruct((NPAD, H), jnp.float32),
    )(xp, encp, W_in[:D_FEAT], W_in[D_FEAT:], b_in[None, :])


def _ee_body(a_ref, w_ref, b_ref, o_ref):
    acc = lax.dot_general(
        a_ref[...], w_ref[...], (((1,), (0,)), ((), ())),
        preferred_element_type=jnp.float32, precision=lax.Precision.HIGHEST)
    acc += b_ref[...]
    o_ref[...] = _leaky(acc)


def _compute_edge_emb(edge_attr, W_edge, b_edge):
    EB = 4000
    grid = (E // EB,)
    return pl.pallas_call(
        _ee_body,
        grid=grid,
        in_specs=[
            pl.BlockSpec((EB, D_EDGE), lambda i: (i, 0)),
            pl.BlockSpec((D_EDGE, H), lambda i: (0, 0)),
            pl.BlockSpec((1, H), lambda i: (0, 0)),
        ],
        out_specs=pl.BlockSpec((EB, H), lambda i: (i, 0)),
        out_shape=jax.ShapeDtypeStruct((E, H), jnp.float32),
    )(edge_attr, W_edge, b_edge[None, :])


def _mlp_body(h_ref, agg_ref, w1_ref, b1_ref, w2_ref, b2_ref, g_ref, be_ref, o_ref):
    v = h_ref[...] + agg_ref[...]
    v = _leaky(lax.dot_general(
        v, w1_ref[...], (((1,), (0,)), ((), ())),
        preferred_element_type=jnp.float32, precision=lax.Precision.HIGHEST) + b1_ref[...])
    v = lax.dot_general(
        v, w2_ref[...], (((1,), (0,)), ((), ())),
        preferred_element_type=jnp.float32, precision=lax.Precision.HIGHEST) + b2_ref[...]
    v = _leaky(v)
    mu = jnp.mean(v, axis=-1, keepdims=True)
    var = jnp.mean((v - mu) ** 2, axis=-1, keepdims=True)
    o_ref[...] = (v - mu) * lax.rsqrt(var + 1e-5) * g_ref[...] + be_ref[...]


def _compute_out(h, agg, W1, b1, W2, b2, ln_g, ln_b):
    NB = 1024
    grid = (NPAD // NB,)
    return pl.pallas_call(
        _mlp_body,
        grid=grid,
        in_specs=[
            pl.BlockSpec((NB, H), lambda i: (i, 0)),
            pl.BlockSpec((NB, H), lambda i: (i, 0)),
            pl.BlockSpec((H, H), lambda i: (0, 0)),
            pl.BlockSpec((1, H), lambda i: (0, 0)),
            pl.BlockSpec((H, H), lambda i: (0, 0)),
            pl.BlockSpec((1, H), lambda i: (0, 0)),
            pl.BlockSpec((1, H), lambda i: (0, 0)),
            pl.BlockSpec((1, H), lambda i: (0, 0)),
        ],
        out_specs=pl.BlockSpec((NB, H), lambda i: (i, 0)),
        out_shape=jax.ShapeDtypeStruct((NPAD, H), jnp.float32),
    )(h, agg, W1, b1[None, :], W2, b2[None, :], ln_g[None, :], ln_b[None, :])


def kernel(x, edge_index, edge_attr, group_emb, W_in, b_in, W_edge, b_edge,
           W1, b1, W2, b2, ln_g, ln_b):
    enc_head = jnp.repeat(group_emb, 16, axis=0)
    encp = jnp.concatenate(
        [enc_head, jnp.zeros((NPAD - 128, EMB), jnp.float32)], axis=0)
    xp = jnp.concatenate([x, jnp.zeros((NPAD - N, D_FEAT), jnp.float32)], axis=0)

    h = _compute_h(xp, encp, W_in, b_in)
    edge_emb = _compute_edge_emb(edge_attr, W_edge, b_edge)

    src = edge_index[0]
    dst = edge_index[1]
    msg = jax.nn.relu(h[src] + edge_emb)
    agg = jnp.zeros((NPAD, H), jnp.float32).at[dst].add(msg)

    out = _compute_out(h, agg, W1, b1, W2, b2, ln_g, ln_b)
    return (out[:N], edge_attr)
